# Initial kernel scaffold; baseline (speedup 1.0000x reference)
import jax
import jax.numpy as jnp
from jax.experimental import pallas as pl


def kernel(token_ids, embed_matrix):
    raise NotImplementedError("write your pallas kernel here")


# speedup vs baseline: 1.8315x; 1.8315x over previous
import functools

import jax
import jax.numpy as jnp
from jax import lax
from jax.experimental import pallas as pl
from jax.experimental.pallas import tpu as pltpu
from jax.experimental.pallas import tpu_sc as plsc

_LANES = 128
_CH = 4
_BLK = _LANES * _CH


def _make_kernel(dim, num_idx):
    info = plsc.get_sparse_core_info()
    nc, ns = info.num_cores, info.num_subcores
    nw = nc * ns
    per_w = num_idx // nw
    p_rows = per_w // _LANES
    nblk = per_w // _BLK

    mesh = plsc.VectorSubcoreMesh(core_axis_name="c", subcore_axis_name="s")

    @functools.partial(
        pl.kernel,
        mesh=mesh,
        out_type=jax.ShapeDtypeStruct((num_idx, dim), jnp.float32),
        compiler_params=pltpu.CompilerParams(use_tc_tiling_on_sc=False),
        scratch_types=[
            pltpu.VMEM((p_rows, _LANES), jnp.int32),
            pltpu.VMEM((_BLK, dim), jnp.float32),
            pltpu.SemaphoreType.DMA,
        ],
    )
    def k(table_hbm, idx_hbm, out_hbm, idx_v, rows_v, sg):
        wid = lax.axis_index("s") * nc + lax.axis_index("c")
        pltpu.sync_copy(idx_hbm.at[pl.ds(wid * p_rows, p_rows)], idx_v)
        base = wid * per_w

        def blk(g, carry):
            for j in range(_CH):
                pltpu.async_copy(
                    table_hbm.at[idx_v.at[g * _CH + j]],
                    rows_v.at[pl.ds(j * _LANES, _LANES)],
                    sg,
                )
            for j in range(_CH):
                pltpu.make_async_copy(
                    table_hbm.at[idx_v.at[g * _CH + j]],
                    rows_v.at[pl.ds(j * _LANES, _LANES)],
                    sg,
                ).wait()
            pltpu.sync_copy(rows_v, out_hbm.at[pl.ds(base + g * _BLK, _BLK)])
            return carry

        lax.fori_loop(0, nblk, blk, 0)

    return k


def kernel(token_ids, embed_matrix):
    b, s = token_ids.shape
    _, d = embed_matrix.shape
    num_idx = b * s
    idx2d = jnp.asarray(token_ids, jnp.int32).reshape(num_idx // _LANES, _LANES)
    out = _make_kernel(d, num_idx)(embed_matrix, idx2d)
    return out.reshape(b, s, d)


# speedup vs baseline: 1.8723x; 1.0223x over previous
import functools

import jax
import jax.numpy as jnp
from jax import lax
from jax.experimental import pallas as pl
from jax.experimental.pallas import tpu as pltpu
from jax.experimental.pallas import tpu_sc as plsc

_LANES = 128
_CH = 4
_BLK = _LANES * _CH


def _make_kernel(dim, num_idx):
    info = plsc.get_sparse_core_info()
    nc, ns = info.num_cores, info.num_subcores
    nw = nc * ns
    per_w = num_idx // nw
    p_rows = per_w // _LANES
    nblk = per_w // _BLK

    mesh = plsc.VectorSubcoreMesh(core_axis_name="c", subcore_axis_name="s")

    @functools.partial(
        pl.kernel,
        mesh=mesh,
        out_type=jax.ShapeDtypeStruct((num_idx, dim), jnp.float32),
        compiler_params=pltpu.CompilerParams(use_tc_tiling_on_sc=False),
        scratch_types=[
            pltpu.VMEM((p_rows, _LANES), jnp.int32),
            pltpu.VMEM((_BLK, dim), jnp.float32),
            pltpu.VMEM((_BLK, dim), jnp.float32),
            pltpu.SemaphoreType.DMA,
            pltpu.SemaphoreType.DMA,
        ],
    )
    def k(table_hbm, idx_hbm, out_hbm, idx_v, rows0, rows1, sg0, sg1):
        wid = lax.axis_index("s") * nc + lax.axis_index("c")
        pltpu.sync_copy(idx_hbm.at[pl.ds(wid * p_rows, p_rows)], idx_v)
        base = wid * per_w

        def fire(g, buf, sem):
            for j in range(_CH):
                pltpu.async_copy(
                    table_hbm.at[idx_v.at[g * _CH + j]],
                    buf.at[pl.ds(j * _LANES, _LANES)],
                    sem,
                )

        def drain(g, buf, sem):
            for j in range(_CH):
                pltpu.make_async_copy(
                    table_hbm.at[idx_v.at[g * _CH + j]],
                    buf.at[pl.ds(j * _LANES, _LANES)],
                    sem,
                ).wait()

        fire(0, rows0, sg0)

        def blk(o, carry):
            g0 = o * 2
            drain(g0, rows0, sg0)
            fire(g0 + 1, rows1, sg1)
            pltpu.sync_copy(rows0, out_hbm.at[pl.ds(base + g0 * _BLK, _BLK)])
            g1 = g0 + 1
            drain(g1, rows1, sg1)

            @pl.when(o < nblk // 2 - 1)
            def _():
                fire(g1 + 1, rows0, sg0)

            pltpu.sync_copy(rows1, out_hbm.at[pl.ds(base + g1 * _BLK, _BLK)])
            return carry

        lax.fori_loop(0, nblk // 2, blk, 0)

    return k


def kernel(token_ids, embed_matrix):
    b, s = token_ids.shape
    _, d = embed_matrix.shape
    num_idx = b * s
    idx2d = jnp.asarray(token_ids, jnp.int32).reshape(num_idx // _LANES, _LANES)
    out = _make_kernel(d, num_idx)(embed_matrix, idx2d)
    return out.reshape(b, s, d)
